# Initial kernel scaffold; baseline (speedup 1.0000x reference)
import jax
import jax.numpy as jnp
from jax.experimental import pallas as pl


def kernel(spk_id, table):
    raise NotImplementedError("write your pallas kernel here")


# speedup vs baseline: 1.3756x; 1.3756x over previous
import jax
import jax.numpy as jnp
from jax import lax
from jax.experimental import pallas as pl
from jax.experimental.pallas import tpu as pltpu
from jax.experimental.pallas import tpu_sc as plsc

SPEAKER_SIZE = 100000
CHANNEL = 256
BATCH = 16384

_NC = 2
_NS = 16
_NW = _NC * _NS
_CHUNK = 128
_PER_W = BATCH // _NW
_NCHUNK = _PER_W // _CHUNK


def _gather_kernel(table_hbm, idx_hbm, out_hbm, idx_v, rows_v, sem0, sem1):
    wid = lax.axis_index("s") * _NC + lax.axis_index("c")
    base = wid * _PER_W
    sems = (sem0, sem1)

    pltpu.sync_copy(idx_hbm.at[wid], idx_v)

    def start_gather(c):
        return pltpu.async_copy(
            table_hbm.at[idx_v.at[c]], rows_v.at[c % 2], sems[c % 2]
        )

    pending = start_gather(0)
    for c in range(_NCHUNK):
        nxt = start_gather(c + 1) if c + 1 < _NCHUNK else None
        pending.wait()
        pltpu.sync_copy(rows_v.at[c % 2],
                        out_hbm.at[pl.ds(base + c * _CHUNK, _CHUNK)])
        pending = nxt


@jax.jit
def kernel(spk_id, table):
    idx3d = spk_id.astype(jnp.int32).reshape(_NW, _NCHUNK, _CHUNK)
    run = pl.kernel(
        _gather_kernel,
        out_type=jax.ShapeDtypeStruct((BATCH, CHANNEL), jnp.float32),
        mesh=plsc.VectorSubcoreMesh(core_axis_name="c", subcore_axis_name="s"),
        scratch_types=[
            pltpu.VMEM((_NCHUNK, _CHUNK), jnp.int32),
            pltpu.VMEM((2, _CHUNK, CHANNEL), jnp.float32),
            pltpu.SemaphoreType.DMA,
            pltpu.SemaphoreType.DMA,
        ],
    )
    return run(table, idx3d)


# speedup vs baseline: 1.3867x; 1.0080x over previous
import jax
import jax.numpy as jnp
from jax import lax
from jax.experimental import pallas as pl
from jax.experimental.pallas import tpu as pltpu
from jax.experimental.pallas import tpu_sc as plsc

SPEAKER_SIZE = 100000
CHANNEL = 256
BATCH = 16384

_NC = 2
_NS = 16
_NW = _NC * _NS
_CHUNK = 128
_PER_W = BATCH // _NW
_NCHUNK = _PER_W // _CHUNK


_NBUF = 3


def _gather_kernel(table_hbm, idx_hbm, out_hbm, idx_v, rows_v,
                   gs0, gs1, gs2, ws0, ws1, ws2):
    wid = lax.axis_index("s") * _NC + lax.axis_index("c")
    base = wid * _PER_W
    gsem = (gs0, gs1, gs2)
    wsem = (ws0, ws1, ws2)

    pltpu.sync_copy(idx_hbm.at[wid], idx_v)

    def start_gather(c):
        b = c % _NBUF
        return pltpu.async_copy(table_hbm.at[idx_v.at[c]], rows_v.at[b],
                                gsem[b])

    def start_write(c):
        b = c % _NBUF
        return pltpu.async_copy(rows_v.at[b],
                                out_hbm.at[pl.ds(base + c * _CHUNK, _CHUNK)],
                                wsem[b])

    g0 = start_gather(0)
    g1 = start_gather(1)
    g2 = start_gather(2)
    g0.wait()
    w0 = start_write(0)
    g1.wait()
    w1 = start_write(1)
    w0.wait()
    g3 = start_gather(3)
    g2.wait()
    w2 = start_write(2)
    g3.wait()
    w3 = start_write(3)
    w1.wait()
    w2.wait()
    w3.wait()


@jax.jit
def kernel(spk_id, table):
    idx3d = spk_id.astype(jnp.int32).reshape(_NW, _NCHUNK, _CHUNK)
    run = pl.kernel(
        _gather_kernel,
        out_type=jax.ShapeDtypeStruct((BATCH, CHANNEL), jnp.float32),
        mesh=plsc.VectorSubcoreMesh(core_axis_name="c", subcore_axis_name="s"),
        scratch_types=[
            pltpu.VMEM((_NCHUNK, _CHUNK), jnp.int32),
            pltpu.VMEM((_NBUF, _CHUNK, CHANNEL), jnp.float32),
            pltpu.SemaphoreType.DMA,
            pltpu.SemaphoreType.DMA,
            pltpu.SemaphoreType.DMA,
            pltpu.SemaphoreType.DMA,
            pltpu.SemaphoreType.DMA,
            pltpu.SemaphoreType.DMA,
        ],
    )
    return run(table, idx3d)


# speedup vs baseline: 1.3880x; 1.0010x over previous
import jax
import jax.numpy as jnp
from jax import lax
from jax.experimental import pallas as pl
from jax.experimental.pallas import tpu as pltpu
from jax.experimental.pallas import tpu_sc as plsc

SPEAKER_SIZE = 100000
CHANNEL = 256
BATCH = 16384

_NC = 2
_NS = 16
_NW = _NC * _NS
_CHUNK = 128
_PER_W = BATCH // _NW
_NCHUNK = _PER_W // _CHUNK


_NBUF = 3


def _gather_kernel(table_hbm, idx_hbm, out_hbm, idx_v, rows_v,
                   gs0, gs1, gs2, ws0, ws1, ws2):
    wid = lax.axis_index("s") * _NC + lax.axis_index("c")
    base = wid * _PER_W
    gsem = (gs0, gs1, gs2)
    wsem = (ws0, ws1, ws2)

    pltpu.sync_copy(idx_hbm.at[pl.ds(base, _PER_W)], idx_v)

    def start_gather(c):
        b = c % _NBUF
        return pltpu.async_copy(
            table_hbm.at[idx_v.at[pl.ds(c * _CHUNK, _CHUNK)]], rows_v.at[b],
            gsem[b])

    def start_write(c):
        b = c % _NBUF
        return pltpu.async_copy(rows_v.at[b],
                                out_hbm.at[pl.ds(base + c * _CHUNK, _CHUNK)],
                                wsem[b])

    g0 = start_gather(0)
    g1 = start_gather(1)
    g2 = start_gather(2)
    g0.wait()
    w0 = start_write(0)
    g1.wait()
    w1 = start_write(1)
    w0.wait()
    g3 = start_gather(3)
    g2.wait()
    w2 = start_write(2)
    g3.wait()
    w3 = start_write(3)
    w1.wait()
    w2.wait()
    w3.wait()


@jax.jit
def kernel(spk_id, table):
    run = pl.kernel(
        _gather_kernel,
        out_type=jax.ShapeDtypeStruct((BATCH, CHANNEL), jnp.float32),
        mesh=plsc.VectorSubcoreMesh(core_axis_name="c", subcore_axis_name="s"),
        scratch_types=[
            pltpu.VMEM((_PER_W,), jnp.int32),
            pltpu.VMEM((_NBUF, _CHUNK, CHANNEL), jnp.float32),
            pltpu.SemaphoreType.DMA,
            pltpu.SemaphoreType.DMA,
            pltpu.SemaphoreType.DMA,
            pltpu.SemaphoreType.DMA,
            pltpu.SemaphoreType.DMA,
            pltpu.SemaphoreType.DMA,
        ],
    )
    return run(table, spk_id)


# speedup vs baseline: 1.4431x; 1.0397x over previous
import jax
import jax.numpy as jnp
from jax import lax
from jax.experimental import pallas as pl
from jax.experimental.pallas import tpu as pltpu
from jax.experimental.pallas import tpu_sc as plsc

SPEAKER_SIZE = 100000
CHANNEL = 256
BATCH = 16384

_NC = 2
_NS = 16
_NW = _NC * _NS
_CHUNK = 64
_PER_W = BATCH // _NW
_NCHUNK = _PER_W // _CHUNK


_NBUF = 7


def _gather_kernel(table_hbm, idx_hbm, out_hbm, idx_v, rows_v, *sems):
    wid = lax.axis_index("s") * _NC + lax.axis_index("c")
    base = wid * _PER_W
    gsem = sems[:_NBUF]
    wsem = sems[_NBUF:]

    pltpu.sync_copy(idx_hbm.at[pl.ds(base, _PER_W)], idx_v)

    def start_gather(c):
        b = c % _NBUF
        return pltpu.async_copy(
            table_hbm.at[idx_v.at[pl.ds(c * _CHUNK, _CHUNK)]], rows_v.at[b],
            gsem[b])

    def start_write(c):
        b = c % _NBUF
        return pltpu.async_copy(rows_v.at[b],
                                out_hbm.at[pl.ds(base + c * _CHUNK, _CHUNK)],
                                wsem[b])

    g = [start_gather(c) for c in range(min(_NBUF, _NCHUNK))]
    g += [None] * (_NCHUNK - len(g))
    w = [None] * _NCHUNK
    for c in range(_NCHUNK):
        nxt = c + _NBUF
        g[c].wait()
        w[c] = start_write(c)
        if nxt < _NCHUNK:
            w[nxt - _NBUF].wait()
            g[nxt] = start_gather(nxt)
    for c in range(max(0, _NCHUNK - _NBUF), _NCHUNK):
        if w[c] is not None:
            w[c].wait()


@jax.jit
def kernel(spk_id, table):
    run = pl.kernel(
        _gather_kernel,
        out_type=jax.ShapeDtypeStruct((BATCH, CHANNEL), jnp.float32),
        mesh=plsc.VectorSubcoreMesh(core_axis_name="c", subcore_axis_name="s"),
        scratch_types=(
            [pltpu.VMEM((_PER_W,), jnp.int32),
             pltpu.VMEM((_NBUF, _CHUNK, CHANNEL), jnp.float32)]
            + [pltpu.SemaphoreType.DMA] * (2 * _NBUF)
        ),
    )
    return run(table, spk_id)


# speedup vs baseline: 1.4463x; 1.0022x over previous
import jax
import jax.numpy as jnp
from jax import lax
from jax.experimental import pallas as pl
from jax.experimental.pallas import tpu as pltpu
from jax.experimental.pallas import tpu_sc as plsc

SPEAKER_SIZE = 100000
CHANNEL = 256
BATCH = 16384

_NC = 2
_NS = 16
_NW = _NC * _NS
_CHUNK = 32
_PER_W = BATCH // _NW
_NCHUNK = _PER_W // _CHUNK


_NBUF = 14


def _gather_kernel(table_hbm, idx_hbm, out_hbm, idx_v, rows_v, *sems):
    wid = lax.axis_index("s") * _NC + lax.axis_index("c")
    base = wid * _PER_W
    gsem = sems[:_NBUF]
    wsem = sems[_NBUF:]

    pltpu.sync_copy(idx_hbm.at[pl.ds(base, _PER_W)], idx_v)

    def start_gather(c):
        b = c % _NBUF
        return pltpu.async_copy(
            table_hbm.at[idx_v.at[pl.ds(c * _CHUNK, _CHUNK)]], rows_v.at[b],
            gsem[b])

    def start_write(c):
        b = c % _NBUF
        return pltpu.async_copy(rows_v.at[b],
                                out_hbm.at[pl.ds(base + c * _CHUNK, _CHUNK)],
                                wsem[b])

    g = [start_gather(c) for c in range(min(_NBUF, _NCHUNK))]
    g += [None] * (_NCHUNK - len(g))
    w = [None] * _NCHUNK
    for c in range(_NCHUNK):
        nxt = c + _NBUF
        g[c].wait()
        w[c] = start_write(c)
        if nxt < _NCHUNK:
            w[nxt - _NBUF].wait()
            g[nxt] = start_gather(nxt)
    for c in range(max(0, _NCHUNK - _NBUF), _NCHUNK):
        if w[c] is not None:
            w[c].wait()


@jax.jit
def kernel(spk_id, table):
    run = pl.kernel(
        _gather_kernel,
        out_type=jax.ShapeDtypeStruct((BATCH, CHANNEL), jnp.float32),
        mesh=plsc.VectorSubcoreMesh(core_axis_name="c", subcore_axis_name="s"),
        scratch_types=(
            [pltpu.VMEM((_PER_W,), jnp.int32),
             pltpu.VMEM((_NBUF, _CHUNK, CHANNEL), jnp.float32)]
            + [pltpu.SemaphoreType.DMA] * (2 * _NBUF)
        ),
    )
    return run(table, spk_id)


# speedup vs baseline: 1.4545x; 1.0057x over previous
import jax
import jax.numpy as jnp
from jax import lax
from jax.experimental import pallas as pl
from jax.experimental.pallas import tpu as pltpu
from jax.experimental.pallas import tpu_sc as plsc

SPEAKER_SIZE = 100000
CHANNEL = 256
BATCH = 16384

_NC = 2
_NS = 16
_NW = _NC * _NS
_CHUNK = 32
_PER_W = BATCH // _NW
_NCHUNK = _PER_W // _CHUNK


_NBUF = 15


def _gather_kernel(table_hbm, idx_hbm, out_hbm, idx_v, rows_v, *sems):
    wid = lax.axis_index("s") * _NC + lax.axis_index("c")
    base = wid * _PER_W
    gsem = sems[:_NBUF]
    wsem = sems[_NBUF:]

    pltpu.sync_copy(idx_hbm.at[pl.ds(base, _PER_W)], idx_v)

    def start_gather(c):
        b = c % _NBUF
        return pltpu.async_copy(
            table_hbm.at[idx_v.at[pl.ds(c * _CHUNK, _CHUNK)]], rows_v.at[b],
            gsem[b])

    def start_write(c):
        b = c % _NBUF
        return pltpu.async_copy(rows_v.at[b],
                                out_hbm.at[pl.ds(base + c * _CHUNK, _CHUNK)],
                                wsem[b])

    g = [start_gather(c) for c in range(min(_NBUF, _NCHUNK))]
    g += [None] * (_NCHUNK - len(g))
    w = [None] * _NCHUNK
    for c in range(_NCHUNK):
        nxt = c + _NBUF
        g[c].wait()
        w[c] = start_write(c)
        if nxt < _NCHUNK:
            w[nxt - _NBUF].wait()
            g[nxt] = start_gather(nxt)
    for c in range(max(0, _NCHUNK - _NBUF), _NCHUNK):
        if w[c] is not None:
            w[c].wait()


@jax.jit
def kernel(spk_id, table):
    run = pl.kernel(
        _gather_kernel,
        out_type=jax.ShapeDtypeStruct((BATCH, CHANNEL), jnp.float32),
        mesh=plsc.VectorSubcoreMesh(core_axis_name="c", subcore_axis_name="s"),
        scratch_types=(
            [pltpu.VMEM((_PER_W,), jnp.int32),
             pltpu.VMEM((_NBUF, _CHUNK, CHANNEL), jnp.float32)]
            + [pltpu.SemaphoreType.DMA] * (2 * _NBUF)
        ),
    )
    return run(table, spk_id)
